# baseline (speedup 1.0000x reference)
import functools

import jax
import jax.numpy as jnp
from jax import lax
from jax.experimental import pallas as pl
from jax.experimental.pallas import tpu as pltpu
from jax.experimental.pallas import tpu_sc as plsc

N_NODES = 10000
N_EDGES = 320000
D = 128

NC = 2
NS = 16
NW = NC * NS

N_PAD = 10240
RPS = N_PAD // NS
TRASH = N_NODES + 64
E_PAD = 327680
EPW = E_PAD // NW
EB = 80
NBLK = EPW // EB
H = D // 2
EPT = E_PAD // NS
NBLKT = EPT // EB

_mesh = plsc.VectorSubcoreMesh(core_axis_name="c", subcore_axis_name="s")


@functools.partial(
    pl.kernel,
    out_type=jax.ShapeDtypeStruct((NC, N_PAD, D), jnp.float32),
    mesh=_mesh,
    scratch_types=[
        pltpu.VMEM((EB,), jnp.int32),
        pltpu.VMEM((EB,), jnp.int32),
        pltpu.VMEM((EB, D), jnp.float32),
        pltpu.VMEM((EB, D), jnp.float32),
        pltpu.VMEM_SHARED((N_PAD, D), jnp.float32),
    ],
)
def _sc_degrees(src_hbm, dst_hbm, zerosD_hbm, onesA_hbm, onesB_hbm, deg_hbm,
                src_v, dst_v, onesA_v, onesB_v, acc):
    c = lax.axis_index("c")
    s = lax.axis_index("s")
    wid = c * NS + s

    pltpu.sync_copy(onesA_hbm, onesA_v)
    pltpu.sync_copy(onesB_hbm, onesB_v)
    pltpu.sync_copy(zerosD_hbm.at[pl.ds(s * RPS, RPS)],
                    acc.at[pl.ds(s * RPS, RPS)])
    plsc.subcore_barrier()

    def block(i, carry):
        base = wid * EPW + i * EB
        pltpu.sync_copy(src_hbm.at[pl.ds(base, EB)], src_v)
        pltpu.sync_copy(dst_hbm.at[pl.ds(base, EB)], dst_v)
        pltpu.sync_copy(onesA_v, acc.at[src_v], add=True)
        pltpu.sync_copy(onesB_v, acc.at[dst_v], add=True)
        return carry
    lax.fori_loop(0, NBLK, block, 0)

    plsc.subcore_barrier()
    pltpu.sync_copy(acc.at[pl.ds(s * RPS, RPS)],
                    deg_hbm.at[c, pl.ds(s * RPS, RPS)])


@functools.partial(
    pl.kernel,
    out_type=jax.ShapeDtypeStruct((NC, N_PAD, D), jnp.float32),
    mesh=_mesh,
    scratch_types=[
        pltpu.VMEM((EB,), jnp.int32),
        pltpu.VMEM((EB,), jnp.int32),
        pltpu.VMEM((EB, D), jnp.float32),
        pltpu.VMEM_SHARED((N_PAD, D), jnp.float32),
        pltpu.SemaphoreType.DMA,
    ],
)
def _sc_aggregate(h_hbm, src_hbm, dst_hbm, zerosD_hbm, out_hbm,
                  src_v, dst_v, rows_v, acc, sem):
    c = lax.axis_index("c")
    s = lax.axis_index("s")
    wid = c * NS + s

    pltpu.sync_copy(zerosD_hbm.at[pl.ds(s * RPS, RPS)],
                    acc.at[pl.ds(s * RPS, RPS)])
    plsc.subcore_barrier()

    def block(i, carry):
        base = wid * EPW + i * EB
        pltpu.sync_copy(src_hbm.at[pl.ds(base, EB)], src_v)
        pltpu.sync_copy(dst_hbm.at[pl.ds(base, EB)], dst_v)
        pltpu.async_copy(h_hbm.at[src_v], rows_v, sem).wait()
        pltpu.sync_copy(rows_v, acc.at[dst_v], add=True)
        return carry
    lax.fori_loop(0, NBLK, block, 0)

    plsc.subcore_barrier()
    pltpu.sync_copy(acc.at[pl.ds(s * RPS, RPS)],
                    out_hbm.at[c, pl.ds(s * RPS, RPS)])


RB = 1280
NRB = N_PAD // RB


def _dinv(deg):
    return jnp.where(deg > 0, lax.rsqrt(jnp.maximum(deg, 1.0)), 0.0)


def _mm(a, w):
    return jnp.dot(a, w, preferred_element_type=jnp.float32,
                   precision=lax.Precision.HIGHEST)


def _tc_pre_body(x, degp, wres, bres, w1,
                 h1_out, res_out, dinv_i_out, dinv_o_out):
    deg = degp[0] + degp[1]
    dinv_o = _dinv(jnp.broadcast_to(deg[:, 0:1], (RB, 16)))
    dinv_i = _dinv(jnp.broadcast_to(deg[:, 64:65], (RB, 16)))
    dinv_o_out[...] = dinv_o
    dinv_i_out[...] = dinv_i
    res_out[...] = _mm(x[...], wres[...]) + bres[...]
    h1_out[...] = _mm(x[...], w1[...]) * dinv_o[:, 0:1]


def _tc_pre(x, deg_p, W_res, b_res, W1):
    return pl.pallas_call(
        _tc_pre_body,
        grid=(NRB,),
        in_specs=[
            pl.BlockSpec((RB, D), lambda r: (r, 0)),
            pl.BlockSpec((NC, RB, D), lambda r: (0, r, 0)),
            pl.BlockSpec((D, D), lambda r: (0, 0)),
            pl.BlockSpec((1, D), lambda r: (0, 0)),
            pl.BlockSpec((D, D), lambda r: (0, 0)),
        ],
        out_specs=[
            pl.BlockSpec((RB, D), lambda r: (r, 0)),
            pl.BlockSpec((RB, D), lambda r: (r, 0)),
            pl.BlockSpec((RB, 16), lambda r: (r, 0)),
            pl.BlockSpec((RB, 16), lambda r: (r, 0)),
        ],
        out_shape=[
            jax.ShapeDtypeStruct((N_PAD, D), jnp.float32),
            jax.ShapeDtypeStruct((N_PAD, D), jnp.float32),
            jax.ShapeDtypeStruct((N_PAD, 16), jnp.float32),
            jax.ShapeDtypeStruct((N_PAD, 16), jnp.float32),
        ],
    )(x, deg_p, W_res, b_res, W1)


def _tc_mid_body(aggp, dinv_i, dinv_o, b_prev, w, h_out):
    agg = aggp[0] + aggp[1]
    z = jnp.maximum(agg * dinv_i[:, 0:1] + b_prev[...], 0.0)
    h_out[...] = _mm(z, w[...]) * dinv_o[:, 0:1]


def _tc_mid(agg_p, dinv_i, dinv_o, b_prev, W_next):
    return pl.pallas_call(
        _tc_mid_body,
        grid=(NRB,),
        in_specs=[
            pl.BlockSpec((NC, RB, D), lambda r: (0, r, 0)),
            pl.BlockSpec((RB, 16), lambda r: (r, 0)),
            pl.BlockSpec((RB, 16), lambda r: (r, 0)),
            pl.BlockSpec((1, D), lambda r: (0, 0)),
            pl.BlockSpec((D, D), lambda r: (0, 0)),
        ],
        out_specs=pl.BlockSpec((RB, D), lambda r: (r, 0)),
        out_shape=jax.ShapeDtypeStruct((N_PAD, D), jnp.float32),
    )(agg_p, dinv_i, dinv_o, b_prev, W_next)


def _tc_post_body(aggp, dinv_i, b4, res, wop, bop, out):
    agg = aggp[0] + aggp[1]
    z = agg * dinv_i[:, 0:1] + b4[...]
    y = jnp.maximum(z + res[...], 0.0)
    out[...] = _mm(y, wop[...]) + bop[...]


def _tc_post(agg_p, dinv_i, b4, res, W_op_pad, b_op_pad):
    return pl.pallas_call(
        _tc_post_body,
        grid=(NRB,),
        in_specs=[
            pl.BlockSpec((NC, RB, D), lambda r: (0, r, 0)),
            pl.BlockSpec((RB, 16), lambda r: (r, 0)),
            pl.BlockSpec((1, D), lambda r: (0, 0)),
            pl.BlockSpec((RB, D), lambda r: (r, 0)),
            pl.BlockSpec((D, D), lambda r: (0, 0)),
            pl.BlockSpec((1, D), lambda r: (0, 0)),
        ],
        out_specs=pl.BlockSpec((RB, D), lambda r: (r, 0)),
        out_shape=jax.ShapeDtypeStruct((N_PAD, D), jnp.float32),
    )(agg_p, dinv_i, b4, res, W_op_pad, b_op_pad)


@jax.jit
def kernel(inputs, edge_index, W_res, b_res, W1, b1, W2, b2, W3, b3, W4, b4,
           W_op, b_op):
    n_classes = W_op.shape[1]
    x = jnp.pad(inputs, ((0, N_PAD - N_NODES), (0, 0)))
    W_op_pad = jnp.pad(W_op, ((0, 0), (0, D - n_classes)))
    b_op_pad = jnp.pad(b_op, ((0, D - n_classes),)).reshape(1, D)
    zerosD = jnp.zeros((N_PAD, D), jnp.float32)
    col = jnp.arange(D)
    onesA = jnp.broadcast_to((col < 64).astype(jnp.float32), (EB, D))
    onesB = jnp.broadcast_to((col >= 64).astype(jnp.float32), (EB, D))
    pad_cfg = ((0, E_PAD - N_EDGES),)
    src = jnp.pad(edge_index[0], pad_cfg, constant_values=TRASH)
    dst = jnp.pad(edge_index[1], pad_cfg, constant_values=TRASH)

    deg_p = _sc_degrees(src, dst, zerosD, onesA, onesB)
    h1, res, dinv_i, dinv_o = _tc_pre(x, deg_p, W_res, b_res.reshape(1, D), W1)

    agg1 = _sc_aggregate(h1, src, dst, zerosD)
    h2 = _tc_mid(agg1, dinv_i, dinv_o, b1.reshape(1, D), W2)
    agg2 = _sc_aggregate(h2, src, dst, zerosD)
    h3 = _tc_mid(agg2, dinv_i, dinv_o, b2.reshape(1, D), W3)
    agg3 = _sc_aggregate(h3, src, dst, zerosD)
    h4 = _tc_mid(agg3, dinv_i, dinv_o, b3.reshape(1, D), W4)
    agg4 = _sc_aggregate(h4, src, dst, zerosD)

    out = _tc_post(agg4, dinv_i, b4.reshape(1, D), res, W_op_pad, b_op_pad)
    return out[:N_NODES, :n_classes]


# speedup vs baseline: 1.6668x; 1.6668x over previous
import functools

import jax
import jax.numpy as jnp
from jax import lax
from jax.experimental import pallas as pl
from jax.experimental.pallas import tpu as pltpu
from jax.experimental.pallas import tpu_sc as plsc

N_NODES = 10000
N_EDGES = 320000
D = 128

NC = 2
NS = 16
NW = NC * NS

N_PAD = 10240
RPS = N_PAD // NS
TRASH = N_NODES + 64
E_PAD = 327680
EPW = E_PAD // NW
EB = 80
NBLK = EPW // EB
H = D // 2
EPT = E_PAD // NS
NBLKT = EPT // EB

_mesh = plsc.VectorSubcoreMesh(core_axis_name="c", subcore_axis_name="s")


@functools.partial(
    pl.kernel,
    out_type=jax.ShapeDtypeStruct((NC, N_PAD, D), jnp.float32),
    mesh=_mesh,
    scratch_types=[
        pltpu.VMEM((EB,), jnp.int32),
        pltpu.VMEM((EB,), jnp.int32),
        pltpu.VMEM((EB, D), jnp.float32),
        pltpu.VMEM((EB, D), jnp.float32),
        pltpu.VMEM_SHARED((N_PAD, D), jnp.float32),
    ],
)
def _sc_degrees(src_hbm, dst_hbm, zerosD_hbm, onesA_hbm, onesB_hbm, deg_hbm,
                src_v, dst_v, onesA_v, onesB_v, acc):
    c = lax.axis_index("c")
    s = lax.axis_index("s")
    wid = c * NS + s

    pltpu.sync_copy(onesA_hbm, onesA_v)
    pltpu.sync_copy(onesB_hbm, onesB_v)
    pltpu.sync_copy(zerosD_hbm.at[pl.ds(s * RPS, RPS)],
                    acc.at[pl.ds(s * RPS, RPS)])
    plsc.subcore_barrier()

    def block(i, carry):
        base = wid * EPW + i * EB
        pltpu.sync_copy(src_hbm.at[pl.ds(base, EB)], src_v)
        pltpu.sync_copy(dst_hbm.at[pl.ds(base, EB)], dst_v)
        pltpu.sync_copy(onesA_v, acc.at[src_v], add=True)
        pltpu.sync_copy(onesB_v, acc.at[dst_v], add=True)
        return carry
    lax.fori_loop(0, NBLK, block, 0)

    plsc.subcore_barrier()
    pltpu.sync_copy(acc.at[pl.ds(s * RPS, RPS)],
                    deg_hbm.at[c, pl.ds(s * RPS, RPS)])


@functools.partial(
    pl.kernel,
    out_type=jax.ShapeDtypeStruct((NC, N_PAD, D), jnp.float32),
    mesh=_mesh,
    scratch_types=[
        pltpu.VMEM((EB,), jnp.int32),
        pltpu.VMEM((EB,), jnp.int32),
        pltpu.VMEM((EB, D), jnp.float32),
        pltpu.VMEM_SHARED((N_PAD, D), jnp.float32),
        pltpu.SemaphoreType.DMA,
    ],
)
def _sc_aggregate(h_hbm, src_hbm, dst_hbm, zerosD_hbm, out_hbm,
                  src_v, dst_v, rows_v, acc, sem):
    c = lax.axis_index("c")
    s = lax.axis_index("s")
    wid = c * NS + s

    pltpu.sync_copy(zerosD_hbm.at[pl.ds(s * RPS, RPS)],
                    acc.at[pl.ds(s * RPS, RPS)])
    plsc.subcore_barrier()

    def block(i, carry):
        base = wid * EPW + i * EB
        pltpu.sync_copy(src_hbm.at[pl.ds(base, EB)], src_v)
        pltpu.sync_copy(dst_hbm.at[pl.ds(base, EB)], dst_v)
        pltpu.async_copy(h_hbm.at[src_v], rows_v, sem).wait()
        pltpu.sync_copy(rows_v, acc.at[dst_v], add=True)
        return carry
    lax.fori_loop(0, NBLK, block, 0)

    plsc.subcore_barrier()
    pltpu.sync_copy(acc.at[pl.ds(s * RPS, RPS)],
                    out_hbm.at[c, pl.ds(s * RPS, RPS)])


RB = 1280
NRB = N_PAD // RB


def _dinv(deg):
    return jnp.where(deg > 0, lax.rsqrt(jnp.maximum(deg, 1.0)), 0.0)


def _mm(a, w):
    return jnp.dot(a, w, preferred_element_type=jnp.float32,
                   precision=lax.Precision.HIGHEST)


def _tc_pre_body(x, degp, wres, bres, w1,
                 h1_out, res_out, dinv_i_out, dinv_o_out):
    deg = degp[0] + degp[1]
    dinv_o = _dinv(jnp.broadcast_to(deg[:, 0:1], (RB, 16)))
    dinv_i = _dinv(jnp.broadcast_to(deg[:, 64:65], (RB, 16)))
    dinv_o_out[...] = dinv_o
    dinv_i_out[...] = dinv_i
    res_out[...] = _mm(x[...], wres[...]) + bres[...]
    h1_out[...] = _mm(x[...], w1[...]) * dinv_o[:, 0:1]


def _tc_pre(x, deg_p, W_res, b_res, W1):
    return pl.pallas_call(
        _tc_pre_body,
        grid=(NRB,),
        in_specs=[
            pl.BlockSpec((RB, D), lambda r: (r, 0)),
            pl.BlockSpec((NC, RB, D), lambda r: (0, r, 0)),
            pl.BlockSpec((D, D), lambda r: (0, 0)),
            pl.BlockSpec((1, D), lambda r: (0, 0)),
            pl.BlockSpec((D, D), lambda r: (0, 0)),
        ],
        out_specs=[
            pl.BlockSpec((RB, D), lambda r: (r, 0)),
            pl.BlockSpec((RB, D), lambda r: (r, 0)),
            pl.BlockSpec((RB, 16), lambda r: (r, 0)),
            pl.BlockSpec((RB, 16), lambda r: (r, 0)),
        ],
        out_shape=[
            jax.ShapeDtypeStruct((N_PAD, D), jnp.float32),
            jax.ShapeDtypeStruct((N_PAD, D), jnp.float32),
            jax.ShapeDtypeStruct((N_PAD, 16), jnp.float32),
            jax.ShapeDtypeStruct((N_PAD, 16), jnp.float32),
        ],
    )(x, deg_p, W_res, b_res, W1)


def _tc_mid_body(aggp, dinv_i, dinv_o, b_prev, w, h_out):
    agg = aggp[0] + aggp[1]
    z = jnp.maximum(agg * dinv_i[:, 0:1] + b_prev[...], 0.0)
    h_out[...] = _mm(z, w[...]) * dinv_o[:, 0:1]


def _tc_mid(agg_p, dinv_i, dinv_o, b_prev, W_next):
    return pl.pallas_call(
        _tc_mid_body,
        grid=(NRB,),
        in_specs=[
            pl.BlockSpec((NC, RB, D), lambda r: (0, r, 0)),
            pl.BlockSpec((RB, 16), lambda r: (r, 0)),
            pl.BlockSpec((RB, 16), lambda r: (r, 0)),
            pl.BlockSpec((1, D), lambda r: (0, 0)),
            pl.BlockSpec((D, D), lambda r: (0, 0)),
        ],
        out_specs=pl.BlockSpec((RB, D), lambda r: (r, 0)),
        out_shape=jax.ShapeDtypeStruct((N_PAD, D), jnp.float32),
    )(agg_p, dinv_i, dinv_o, b_prev, W_next)


def _tc_post_body(aggp, dinv_i, b4, res, wop, bop, out):
    agg = aggp[0] + aggp[1]
    z = agg * dinv_i[:, 0:1] + b4[...]
    y = jnp.maximum(z + res[...], 0.0)
    out[...] = _mm(y, wop[...]) + bop[...]


def _tc_post(agg_p, dinv_i, b4, res, W_op_pad, b_op_pad):
    return pl.pallas_call(
        _tc_post_body,
        grid=(NRB,),
        in_specs=[
            pl.BlockSpec((NC, RB, D), lambda r: (0, r, 0)),
            pl.BlockSpec((RB, 16), lambda r: (r, 0)),
            pl.BlockSpec((1, D), lambda r: (0, 0)),
            pl.BlockSpec((RB, D), lambda r: (r, 0)),
            pl.BlockSpec((D, D), lambda r: (0, 0)),
            pl.BlockSpec((1, D), lambda r: (0, 0)),
        ],
        out_specs=pl.BlockSpec((RB, D), lambda r: (r, 0)),
        out_shape=jax.ShapeDtypeStruct((N_PAD, D), jnp.float32),
    )(agg_p, dinv_i, b4, res, W_op_pad, b_op_pad)


@jax.jit
def kernel(inputs, edge_index, W_res, b_res, W1, b1, W2, b2, W3, b3, W4, b4,
           W_op, b_op):
    n_classes = W_op.shape[1]
    x = jnp.pad(inputs, ((0, N_PAD - N_NODES), (0, 0)))
    W_op_pad = jnp.pad(W_op, ((0, 0), (0, D - n_classes)))
    b_op_pad = jnp.pad(b_op, ((0, D - n_classes),)).reshape(1, D)
    zerosD = jnp.zeros((N_PAD, D), jnp.float32)
    col = jnp.arange(D)
    onesA = jnp.broadcast_to((col < 64).astype(jnp.float32), (EB, D))
    onesB = jnp.broadcast_to((col >= 64).astype(jnp.float32), (EB, D))
    pad_idx = (N_NODES
               + jnp.arange(E_PAD - N_EDGES, dtype=jnp.int32)
               % (N_PAD - N_NODES))
    src = jnp.concatenate([edge_index[0], pad_idx])
    dst = jnp.concatenate([edge_index[1], pad_idx])

    deg_p = _sc_degrees(src, dst, zerosD, onesA, onesB)
    h1, res, dinv_i, dinv_o = _tc_pre(x, deg_p, W_res, b_res.reshape(1, D), W1)

    agg1 = _sc_aggregate(h1, src, dst, zerosD)
    h2 = _tc_mid(agg1, dinv_i, dinv_o, b1.reshape(1, D), W2)
    agg2 = _sc_aggregate(h2, src, dst, zerosD)
    h3 = _tc_mid(agg2, dinv_i, dinv_o, b2.reshape(1, D), W3)
    agg3 = _sc_aggregate(h3, src, dst, zerosD)
    h4 = _tc_mid(agg3, dinv_i, dinv_o, b3.reshape(1, D), W4)
    agg4 = _sc_aggregate(h4, src, dst, zerosD)

    out = _tc_post(agg4, dinv_i, b4.reshape(1, D), res, W_op_pad, b_op_pad)
    return out[:N_NODES, :n_classes]


# speedup vs baseline: 2.4507x; 1.4703x over previous
import functools

import jax
import jax.numpy as jnp
from jax import lax
from jax.experimental import pallas as pl
from jax.experimental.pallas import tpu as pltpu
from jax.experimental.pallas import tpu_sc as plsc

N_NODES = 10000
N_EDGES = 320000
D = 128

NC = 2
NS = 16
NW = NC * NS

N_PAD = 10240
RPS = N_PAD // NS
TRASH = N_NODES + 64
E_PAD = 327680
EPW = E_PAD // NW
EB = 80
NBLK = EPW // EB
H = D // 2
EPT = E_PAD // NS
NBLKT = EPT // EB

_mesh = plsc.VectorSubcoreMesh(core_axis_name="c", subcore_axis_name="s")


@functools.partial(
    pl.kernel,
    out_type=jax.ShapeDtypeStruct((NC, N_PAD, D), jnp.float32),
    mesh=_mesh,
    scratch_types=[
        pltpu.VMEM((EB,), jnp.int32),
        pltpu.VMEM((EB,), jnp.int32),
        pltpu.VMEM((EB, D), jnp.float32),
        pltpu.VMEM((EB, D), jnp.float32),
        pltpu.VMEM_SHARED((N_PAD, D), jnp.float32),
    ],
)
def _sc_degrees(src_hbm, dst_hbm, zerosD_hbm, onesA_hbm, onesB_hbm, deg_hbm,
                src_v, dst_v, onesA_v, onesB_v, acc):
    c = lax.axis_index("c")
    s = lax.axis_index("s")
    wid = c * NS + s

    pltpu.sync_copy(onesA_hbm, onesA_v)
    pltpu.sync_copy(onesB_hbm, onesB_v)
    pltpu.sync_copy(zerosD_hbm.at[pl.ds(s * RPS, RPS)],
                    acc.at[pl.ds(s * RPS, RPS)])
    plsc.subcore_barrier()

    def block(i, carry):
        base = wid * EPW + i * EB
        pltpu.sync_copy(src_hbm.at[pl.ds(base, EB)], src_v)
        pltpu.sync_copy(dst_hbm.at[pl.ds(base, EB)], dst_v)
        pltpu.sync_copy(onesA_v, acc.at[src_v], add=True)
        pltpu.sync_copy(onesB_v, acc.at[dst_v], add=True)
        return carry
    lax.fori_loop(0, NBLK, block, 0)

    plsc.subcore_barrier()
    pltpu.sync_copy(acc.at[pl.ds(s * RPS, RPS)],
                    deg_hbm.at[c, pl.ds(s * RPS, RPS)])


@functools.partial(
    pl.kernel,
    out_type=jax.ShapeDtypeStruct((NC, N_PAD, D), jnp.float32),
    mesh=_mesh,
    scratch_types=[
        pltpu.VMEM((EB,), jnp.int32),
        pltpu.VMEM((EB,), jnp.int32),
        pltpu.VMEM((EB,), jnp.int32),
        pltpu.VMEM((EB,), jnp.int32),
        pltpu.VMEM((EB, D), jnp.float32),
        pltpu.VMEM((EB, D), jnp.float32),
        pltpu.VMEM_SHARED((N_PAD, D), jnp.float32),
        pltpu.SemaphoreType.DMA,
        pltpu.SemaphoreType.DMA,
    ],
)
def _sc_aggregate(h_hbm, src_hbm, dst_hbm, zerosD_hbm, out_hbm,
                  src_v0, dst_v0, src_v1, dst_v1, rows0, rows1, acc,
                  gsem0, gsem1):
    c = lax.axis_index("c")
    s = lax.axis_index("s")
    wid = c * NS + s
    ebase = wid * EPW

    pltpu.sync_copy(zerosD_hbm.at[pl.ds(s * RPS, RPS)],
                    acc.at[pl.ds(s * RPS, RPS)])
    plsc.subcore_barrier()

    def idx(i, sv, dv):
        pltpu.sync_copy(src_hbm.at[pl.ds(ebase + i * EB, EB)], sv)
        pltpu.sync_copy(dst_hbm.at[pl.ds(ebase + i * EB, EB)], dv)

    def gather(sv, buf, sem):
        return pltpu.async_copy(h_hbm.at[sv], buf, sem)

    def drain(sv, buf, sem):
        pltpu.make_async_copy(h_hbm.at[sv], buf, sem).wait()

    def scatter(dv, buf):
        pltpu.sync_copy(buf, acc.at[dv], add=True)

    idx(0, src_v0, dst_v0)
    gather(src_v0, rows0, gsem0)

    def pair(k, carry):
        b0 = 2 * k
        idx(b0 + 1, src_v1, dst_v1)
        drain(src_v0, rows0, gsem0)
        gather(src_v1, rows1, gsem1)
        scatter(dst_v0, rows0)
        idx(b0 + 2, src_v0, dst_v0)
        drain(src_v1, rows1, gsem1)
        gather(src_v0, rows0, gsem0)
        scatter(dst_v1, rows1)
        return carry
    lax.fori_loop(0, NBLK // 2 - 1, pair, 0)

    idx(NBLK - 1, src_v1, dst_v1)
    drain(src_v0, rows0, gsem0)
    gather(src_v1, rows1, gsem1)
    scatter(dst_v0, rows0)
    drain(src_v1, rows1, gsem1)
    scatter(dst_v1, rows1)

    plsc.subcore_barrier()
    pltpu.sync_copy(acc.at[pl.ds(s * RPS, RPS)],
                    out_hbm.at[c, pl.ds(s * RPS, RPS)])


RB = 1280
NRB = N_PAD // RB


def _dinv(deg):
    return jnp.where(deg > 0, lax.rsqrt(jnp.maximum(deg, 1.0)), 0.0)


def _mm(a, w):
    return jnp.dot(a, w, preferred_element_type=jnp.float32,
                   precision=lax.Precision.HIGHEST)


def _tc_pre_body(x, degp, wres, bres, w1,
                 h1_out, res_out, dinv_i_out, dinv_o_out):
    deg = degp[0] + degp[1]
    dinv_o = _dinv(jnp.broadcast_to(deg[:, 0:1], (RB, 16)))
    dinv_i = _dinv(jnp.broadcast_to(deg[:, 64:65], (RB, 16)))
    dinv_o_out[...] = dinv_o
    dinv_i_out[...] = dinv_i
    res_out[...] = _mm(x[...], wres[...]) + bres[...]
    h1_out[...] = _mm(x[...], w1[...]) * dinv_o[:, 0:1]


def _tc_pre(x, deg_p, W_res, b_res, W1):
    return pl.pallas_call(
        _tc_pre_body,
        grid=(NRB,),
        in_specs=[
            pl.BlockSpec((RB, D), lambda r: (r, 0)),
            pl.BlockSpec((NC, RB, D), lambda r: (0, r, 0)),
            pl.BlockSpec((D, D), lambda r: (0, 0)),
            pl.BlockSpec((1, D), lambda r: (0, 0)),
            pl.BlockSpec((D, D), lambda r: (0, 0)),
        ],
        out_specs=[
            pl.BlockSpec((RB, D), lambda r: (r, 0)),
            pl.BlockSpec((RB, D), lambda r: (r, 0)),
            pl.BlockSpec((RB, 16), lambda r: (r, 0)),
            pl.BlockSpec((RB, 16), lambda r: (r, 0)),
        ],
        out_shape=[
            jax.ShapeDtypeStruct((N_PAD, D), jnp.float32),
            jax.ShapeDtypeStruct((N_PAD, D), jnp.float32),
            jax.ShapeDtypeStruct((N_PAD, 16), jnp.float32),
            jax.ShapeDtypeStruct((N_PAD, 16), jnp.float32),
        ],
    )(x, deg_p, W_res, b_res, W1)


def _tc_mid_body(aggp, dinv_i, dinv_o, b_prev, w, h_out):
    agg = aggp[0] + aggp[1]
    z = jnp.maximum(agg * dinv_i[:, 0:1] + b_prev[...], 0.0)
    h_out[...] = _mm(z, w[...]) * dinv_o[:, 0:1]


def _tc_mid(agg_p, dinv_i, dinv_o, b_prev, W_next):
    return pl.pallas_call(
        _tc_mid_body,
        grid=(NRB,),
        in_specs=[
            pl.BlockSpec((NC, RB, D), lambda r: (0, r, 0)),
            pl.BlockSpec((RB, 16), lambda r: (r, 0)),
            pl.BlockSpec((RB, 16), lambda r: (r, 0)),
            pl.BlockSpec((1, D), lambda r: (0, 0)),
            pl.BlockSpec((D, D), lambda r: (0, 0)),
        ],
        out_specs=pl.BlockSpec((RB, D), lambda r: (r, 0)),
        out_shape=jax.ShapeDtypeStruct((N_PAD, D), jnp.float32),
    )(agg_p, dinv_i, dinv_o, b_prev, W_next)


def _tc_post_body(aggp, dinv_i, b4, res, wop, bop, out):
    agg = aggp[0] + aggp[1]
    z = agg * dinv_i[:, 0:1] + b4[...]
    y = jnp.maximum(z + res[...], 0.0)
    out[...] = _mm(y, wop[...]) + bop[...]


def _tc_post(agg_p, dinv_i, b4, res, W_op_pad, b_op_pad):
    return pl.pallas_call(
        _tc_post_body,
        grid=(NRB,),
        in_specs=[
            pl.BlockSpec((NC, RB, D), lambda r: (0, r, 0)),
            pl.BlockSpec((RB, 16), lambda r: (r, 0)),
            pl.BlockSpec((1, D), lambda r: (0, 0)),
            pl.BlockSpec((RB, D), lambda r: (r, 0)),
            pl.BlockSpec((D, D), lambda r: (0, 0)),
            pl.BlockSpec((1, D), lambda r: (0, 0)),
        ],
        out_specs=pl.BlockSpec((RB, D), lambda r: (r, 0)),
        out_shape=jax.ShapeDtypeStruct((N_PAD, D), jnp.float32),
    )(agg_p, dinv_i, b4, res, W_op_pad, b_op_pad)


@jax.jit
def kernel(inputs, edge_index, W_res, b_res, W1, b1, W2, b2, W3, b3, W4, b4,
           W_op, b_op):
    n_classes = W_op.shape[1]
    x = jnp.pad(inputs, ((0, N_PAD - N_NODES), (0, 0)))
    W_op_pad = jnp.pad(W_op, ((0, 0), (0, D - n_classes)))
    b_op_pad = jnp.pad(b_op, ((0, D - n_classes),)).reshape(1, D)
    zerosD = jnp.zeros((N_PAD, D), jnp.float32)
    col = jnp.arange(D)
    onesA = jnp.broadcast_to((col < 64).astype(jnp.float32), (EB, D))
    onesB = jnp.broadcast_to((col >= 64).astype(jnp.float32), (EB, D))
    pad_idx = (N_NODES
               + jnp.arange(E_PAD - N_EDGES, dtype=jnp.int32)
               % (N_PAD - N_NODES))
    src = jnp.concatenate([edge_index[0], pad_idx])
    dst = jnp.concatenate([edge_index[1], pad_idx])

    deg_p = _sc_degrees(src, dst, zerosD, onesA, onesB)
    h1, res, dinv_i, dinv_o = _tc_pre(x, deg_p, W_res, b_res.reshape(1, D), W1)

    agg1 = _sc_aggregate(h1, src, dst, zerosD)
    h2 = _tc_mid(agg1, dinv_i, dinv_o, b1.reshape(1, D), W2)
    agg2 = _sc_aggregate(h2, src, dst, zerosD)
    h3 = _tc_mid(agg2, dinv_i, dinv_o, b2.reshape(1, D), W3)
    agg3 = _sc_aggregate(h3, src, dst, zerosD)
    h4 = _tc_mid(agg3, dinv_i, dinv_o, b3.reshape(1, D), W4)
    agg4 = _sc_aggregate(h4, src, dst, zerosD)

    out = _tc_post(agg4, dinv_i, b4.reshape(1, D), res, W_op_pad, b_op_pad)
    return out[:N_NODES, :n_classes]


# speedup vs baseline: 2.4688x; 1.0074x over previous
import functools

import jax
import jax.numpy as jnp
from jax import lax
from jax.experimental import pallas as pl
from jax.experimental.pallas import tpu as pltpu
from jax.experimental.pallas import tpu_sc as plsc

N_NODES = 10000
N_EDGES = 320000
D = 128

NC = 2
NS = 16
NW = NC * NS

N_PAD = 10240
RPS = N_PAD // NS
TRASH = N_NODES + 64
E_PAD = 327680
EPW = E_PAD // NW
EB = 80
NBLK = EPW // EB
H = D // 2
EPT = E_PAD // NS
NBLKT = EPT // EB

_mesh = plsc.VectorSubcoreMesh(core_axis_name="c", subcore_axis_name="s")


@functools.partial(
    pl.kernel,
    out_type=jax.ShapeDtypeStruct((NC, N_PAD, D), jnp.float32),
    mesh=_mesh,
    scratch_types=[
        pltpu.VMEM((EB,), jnp.int32),
        pltpu.VMEM((EB,), jnp.int32),
        pltpu.VMEM((EB,), jnp.int32),
        pltpu.VMEM((EB,), jnp.int32),
        pltpu.VMEM((EB, D), jnp.float32),
        pltpu.VMEM((EB, D), jnp.float32),
        pltpu.VMEM_SHARED((N_PAD, D), jnp.float32),
        pltpu.SemaphoreType.DMA,
    ],
)
def _sc_degrees(src_hbm, dst_hbm, zerosD_hbm, onesA_hbm, onesB_hbm, deg_hbm,
                src_v0, dst_v0, src_v1, dst_v1, onesA_v, onesB_v, acc, sem):
    c = lax.axis_index("c")
    s = lax.axis_index("s")
    wid = c * NS + s
    ebase = wid * EPW

    pltpu.sync_copy(onesA_hbm, onesA_v)
    pltpu.sync_copy(onesB_hbm, onesB_v)
    pltpu.sync_copy(zerosD_hbm.at[pl.ds(s * RPS, RPS)],
                    acc.at[pl.ds(s * RPS, RPS)])
    plsc.subcore_barrier()

    def idx(i, sv, dv):
        pltpu.sync_copy(src_hbm.at[pl.ds(ebase + i * EB, EB)], sv)
        pltpu.sync_copy(dst_hbm.at[pl.ds(ebase + i * EB, EB)], dv)

    def scat(sv, dv):
        d = pltpu.async_copy(onesA_v, acc.at[sv], sem, add=True)
        pltpu.sync_copy(onesB_v, acc.at[dv], add=True)
        d.wait()

    idx(0, src_v0, dst_v0)

    def pair(k, carry):
        b0 = 2 * k
        idx(b0 + 1, src_v1, dst_v1)
        scat(src_v0, dst_v0)
        idx(b0 + 2, src_v0, dst_v0)
        scat(src_v1, dst_v1)
        return carry
    lax.fori_loop(0, NBLK // 2 - 1, pair, 0)

    idx(NBLK - 1, src_v1, dst_v1)
    scat(src_v0, dst_v0)
    scat(src_v1, dst_v1)

    plsc.subcore_barrier()
    pltpu.sync_copy(acc.at[pl.ds(s * RPS, RPS)],
                    deg_hbm.at[c, pl.ds(s * RPS, RPS)])


@functools.partial(
    pl.kernel,
    out_type=jax.ShapeDtypeStruct((NC, N_PAD, D), jnp.float32),
    mesh=_mesh,
    scratch_types=[
        pltpu.VMEM((EB,), jnp.int32),
        pltpu.VMEM((EB,), jnp.int32),
        pltpu.VMEM((EB,), jnp.int32),
        pltpu.VMEM((EB,), jnp.int32),
        pltpu.VMEM((EB, D), jnp.float32),
        pltpu.VMEM((EB, D), jnp.float32),
        pltpu.VMEM_SHARED((N_PAD, D), jnp.float32),
        pltpu.SemaphoreType.DMA,
        pltpu.SemaphoreType.DMA,
    ],
)
def _sc_aggregate(h_hbm, src_hbm, dst_hbm, zerosD_hbm, out_hbm,
                  src_v0, dst_v0, src_v1, dst_v1, rows0, rows1, acc,
                  gsem0, gsem1):
    c = lax.axis_index("c")
    s = lax.axis_index("s")
    wid = c * NS + s
    ebase = wid * EPW

    pltpu.sync_copy(zerosD_hbm.at[pl.ds(s * RPS, RPS)],
                    acc.at[pl.ds(s * RPS, RPS)])
    plsc.subcore_barrier()

    def idx(i, sv, dv):
        pltpu.sync_copy(src_hbm.at[pl.ds(ebase + i * EB, EB)], sv)
        pltpu.sync_copy(dst_hbm.at[pl.ds(ebase + i * EB, EB)], dv)

    def gather(sv, buf, sem):
        return pltpu.async_copy(h_hbm.at[sv], buf, sem)

    def drain(sv, buf, sem):
        pltpu.make_async_copy(h_hbm.at[sv], buf, sem).wait()

    def scatter(dv, buf):
        pltpu.sync_copy(buf, acc.at[dv], add=True)

    idx(0, src_v0, dst_v0)
    gather(src_v0, rows0, gsem0)

    def pair(k, carry):
        b0 = 2 * k
        idx(b0 + 1, src_v1, dst_v1)
        drain(src_v0, rows0, gsem0)
        gather(src_v1, rows1, gsem1)
        scatter(dst_v0, rows0)
        idx(b0 + 2, src_v0, dst_v0)
        drain(src_v1, rows1, gsem1)
        gather(src_v0, rows0, gsem0)
        scatter(dst_v1, rows1)
        return carry
    lax.fori_loop(0, NBLK // 2 - 1, pair, 0)

    idx(NBLK - 1, src_v1, dst_v1)
    drain(src_v0, rows0, gsem0)
    gather(src_v1, rows1, gsem1)
    scatter(dst_v0, rows0)
    drain(src_v1, rows1, gsem1)
    scatter(dst_v1, rows1)

    plsc.subcore_barrier()
    pltpu.sync_copy(acc.at[pl.ds(s * RPS, RPS)],
                    out_hbm.at[c, pl.ds(s * RPS, RPS)])


RB = 1280
NRB = N_PAD // RB


def _dinv(deg):
    return jnp.where(deg > 0, lax.rsqrt(jnp.maximum(deg, 1.0)), 0.0)


def _mm(a, w):
    return jnp.dot(a, w, preferred_element_type=jnp.float32,
                   precision=lax.Precision.HIGHEST)


def _tc_pre_body(x, degp, wres, bres, w1,
                 h1_out, res_out, dinv_i_out, dinv_o_out):
    deg = degp[0] + degp[1]
    dinv_o = _dinv(jnp.broadcast_to(deg[:, 0:1], (RB, 16)))
    dinv_i = _dinv(jnp.broadcast_to(deg[:, 64:65], (RB, 16)))
    dinv_o_out[...] = dinv_o
    dinv_i_out[...] = dinv_i
    res_out[...] = _mm(x[...], wres[...]) + bres[...]
    h1_out[...] = _mm(x[...], w1[...]) * dinv_o[:, 0:1]


def _tc_pre(x, deg_p, W_res, b_res, W1):
    return pl.pallas_call(
        _tc_pre_body,
        grid=(NRB,),
        in_specs=[
            pl.BlockSpec((RB, D), lambda r: (r, 0)),
            pl.BlockSpec((NC, RB, D), lambda r: (0, r, 0)),
            pl.BlockSpec((D, D), lambda r: (0, 0)),
            pl.BlockSpec((1, D), lambda r: (0, 0)),
            pl.BlockSpec((D, D), lambda r: (0, 0)),
        ],
        out_specs=[
            pl.BlockSpec((RB, D), lambda r: (r, 0)),
            pl.BlockSpec((RB, D), lambda r: (r, 0)),
            pl.BlockSpec((RB, 16), lambda r: (r, 0)),
            pl.BlockSpec((RB, 16), lambda r: (r, 0)),
        ],
        out_shape=[
            jax.ShapeDtypeStruct((N_PAD, D), jnp.float32),
            jax.ShapeDtypeStruct((N_PAD, D), jnp.float32),
            jax.ShapeDtypeStruct((N_PAD, 16), jnp.float32),
            jax.ShapeDtypeStruct((N_PAD, 16), jnp.float32),
        ],
    )(x, deg_p, W_res, b_res, W1)


def _tc_mid_body(aggp, dinv_i, dinv_o, b_prev, w, h_out):
    agg = aggp[0] + aggp[1]
    z = jnp.maximum(agg * dinv_i[:, 0:1] + b_prev[...], 0.0)
    h_out[...] = _mm(z, w[...]) * dinv_o[:, 0:1]


def _tc_mid(agg_p, dinv_i, dinv_o, b_prev, W_next):
    return pl.pallas_call(
        _tc_mid_body,
        grid=(NRB,),
        in_specs=[
            pl.BlockSpec((NC, RB, D), lambda r: (0, r, 0)),
            pl.BlockSpec((RB, 16), lambda r: (r, 0)),
            pl.BlockSpec((RB, 16), lambda r: (r, 0)),
            pl.BlockSpec((1, D), lambda r: (0, 0)),
            pl.BlockSpec((D, D), lambda r: (0, 0)),
        ],
        out_specs=pl.BlockSpec((RB, D), lambda r: (r, 0)),
        out_shape=jax.ShapeDtypeStruct((N_PAD, D), jnp.float32),
    )(agg_p, dinv_i, dinv_o, b_prev, W_next)


def _tc_post_body(aggp, dinv_i, b4, res, wop, bop, out):
    agg = aggp[0] + aggp[1]
    z = agg * dinv_i[:, 0:1] + b4[...]
    y = jnp.maximum(z + res[...], 0.0)
    out[...] = _mm(y, wop[...]) + bop[...]


def _tc_post(agg_p, dinv_i, b4, res, W_op_pad, b_op_pad):
    return pl.pallas_call(
        _tc_post_body,
        grid=(NRB,),
        in_specs=[
            pl.BlockSpec((NC, RB, D), lambda r: (0, r, 0)),
            pl.BlockSpec((RB, 16), lambda r: (r, 0)),
            pl.BlockSpec((1, D), lambda r: (0, 0)),
            pl.BlockSpec((RB, D), lambda r: (r, 0)),
            pl.BlockSpec((D, D), lambda r: (0, 0)),
            pl.BlockSpec((1, D), lambda r: (0, 0)),
        ],
        out_specs=pl.BlockSpec((RB, D), lambda r: (r, 0)),
        out_shape=jax.ShapeDtypeStruct((N_PAD, D), jnp.float32),
    )(agg_p, dinv_i, b4, res, W_op_pad, b_op_pad)


@jax.jit
def kernel(inputs, edge_index, W_res, b_res, W1, b1, W2, b2, W3, b3, W4, b4,
           W_op, b_op):
    n_classes = W_op.shape[1]
    x = jnp.pad(inputs, ((0, N_PAD - N_NODES), (0, 0)))
    W_op_pad = jnp.pad(W_op, ((0, 0), (0, D - n_classes)))
    b_op_pad = jnp.pad(b_op, ((0, D - n_classes),)).reshape(1, D)
    zerosD = jnp.zeros((N_PAD, D), jnp.float32)
    col = jnp.arange(D)
    onesA = jnp.broadcast_to((col < 64).astype(jnp.float32), (EB, D))
    onesB = jnp.broadcast_to((col >= 64).astype(jnp.float32), (EB, D))
    pad_idx = (N_NODES
               + jnp.arange(E_PAD - N_EDGES, dtype=jnp.int32)
               % (N_PAD - N_NODES))
    src = jnp.concatenate([edge_index[0], pad_idx])
    dst = jnp.concatenate([edge_index[1], pad_idx])

    deg_p = _sc_degrees(src, dst, zerosD, onesA, onesB)
    h1, res, dinv_i, dinv_o = _tc_pre(x, deg_p, W_res, b_res.reshape(1, D), W1)

    agg1 = _sc_aggregate(h1, src, dst, zerosD)
    h2 = _tc_mid(agg1, dinv_i, dinv_o, b1.reshape(1, D), W2)
    agg2 = _sc_aggregate(h2, src, dst, zerosD)
    h3 = _tc_mid(agg2, dinv_i, dinv_o, b2.reshape(1, D), W3)
    agg3 = _sc_aggregate(h3, src, dst, zerosD)
    h4 = _tc_mid(agg3, dinv_i, dinv_o, b3.reshape(1, D), W4)
    agg4 = _sc_aggregate(h4, src, dst, zerosD)

    out = _tc_post(agg4, dinv_i, b4.reshape(1, D), res, W_op_pad, b_op_pad)
    return out[:N_NODES, :n_classes]
